# baseline (speedup 1.0000x reference)
import jax
import jax.numpy as jnp
from jax.experimental import pallas as pl
from jax.experimental.pallas import tpu as pltpu

_HW_BLK = 512


def _kwc_block(x_ref, o_ref):
    xb = x_ref[...]
    C = xb.shape[0]
    ones = jnp.ones((1, C), dtype=xb.dtype)
    s = jnp.dot(ones, xb, preferred_element_type=jnp.float32)
    m = s * (1.0 / C)
    o_ref[...] = jnp.maximum(xb - m, 0.0)


def kernel(x, k):
    del k
    B, C, H, W = x.shape
    HW = H * W
    x3 = x.reshape(B * C, HW)
    grid = (B, HW // _HW_BLK)
    out = pl.pallas_call(
        _kwc_block,
        grid=grid,
        in_specs=[pl.BlockSpec((C, _HW_BLK), lambda b, j: (b, j))],
        out_specs=pl.BlockSpec((C, _HW_BLK), lambda b, j: (b, j)),
        out_shape=jax.ShapeDtypeStruct((B * C, HW), x.dtype),
        compiler_params=pltpu.CompilerParams(
            dimension_semantics=("parallel", "parallel"),
        ),
    )(x3)
    return out.reshape(B, C, H, W)


# speedup vs baseline: 1.0157x; 1.0157x over previous
import jax
import jax.numpy as jnp
from jax.experimental import pallas as pl
from jax.experimental.pallas import tpu as pltpu

_HW_BLK = 512


def _kwc_block(x_ref, o_ref):
    xb = x_ref[...]
    C = xb.shape[0]
    m = jnp.sum(xb, axis=0, keepdims=True) * (1.0 / C)
    o_ref[...] = jnp.maximum(xb - m, 0.0)


def kernel(x, k):
    del k
    B, C, H, W = x.shape
    HW = H * W
    x3 = x.reshape(B * C, HW)
    grid = (B, HW // _HW_BLK)
    out = pl.pallas_call(
        _kwc_block,
        grid=grid,
        in_specs=[pl.BlockSpec((C, _HW_BLK), lambda b, j: (b, j))],
        out_specs=pl.BlockSpec((C, _HW_BLK), lambda b, j: (b, j)),
        out_shape=jax.ShapeDtypeStruct((B * C, HW), x.dtype),
        compiler_params=pltpu.CompilerParams(
            dimension_semantics=("parallel", "parallel"),
        ),
    )(x3)
    return out.reshape(B, C, H, W)


# speedup vs baseline: 2.4308x; 2.3932x over previous
import jax
import jax.numpy as jnp
from jax.experimental import pallas as pl
from jax.experimental.pallas import tpu as pltpu

_HW_BLK = 512


def _kwc_block(x_ref, o_ref):
    xb = x_ref[...]
    C = xb.shape[1]
    m = jnp.sum(xb, axis=1, keepdims=True) * (1.0 / C)
    o_ref[...] = jnp.maximum(xb - m, 0.0)


def kernel(x, k):
    del k
    B, C, H, W = x.shape
    HW = H * W
    x3 = x.reshape(B, C, HW)
    out = pl.pallas_call(
        _kwc_block,
        grid=(B,),
        in_specs=[pl.BlockSpec((1, C, HW), lambda b: (b, 0, 0))],
        out_specs=pl.BlockSpec((1, C, HW), lambda b: (b, 0, 0)),
        out_shape=jax.ShapeDtypeStruct((B, C, HW), x.dtype),
        compiler_params=pltpu.CompilerParams(
            dimension_semantics=("parallel",),
        ),
    )(x3)
    return out.reshape(B, C, H, W)
